# baseline (speedup 1.0000x reference)
import functools

import jax
import jax.numpy as jnp
from jax import lax
from jax.experimental import pallas as pl
from jax.experimental.pallas import tpu as pltpu
from jax.experimental.pallas import tpu_sc as plsc

H = 256
NC, NS = 2, 16
NW = NC * NS
C = 64
NBUF = 4


@functools.partial(jax.jit, static_argnums=(2, 3))
def _sc_lookup(ids, table_flat, BT, S):
    n_per_w = BT // NW
    nchunks = n_per_w // C
    rows_per_chunk = C // S
    mesh = plsc.VectorSubcoreMesh(
        core_axis_name="c", subcore_axis_name="s",
        num_cores=NC, num_subcores=NS,
    )

    @functools.partial(
        pl.kernel,
        out_type=jax.ShapeDtypeStruct((BT // S, S, 1, H), jnp.float32),
        mesh=mesh,
        compiler_params=pltpu.CompilerParams(needs_layout_passes=False),
        scratch_types=[
            pltpu.VMEM((H * 3,), jnp.float32),
            pltpu.VMEM((n_per_w,), jnp.int32),
        ] + [pltpu.VMEM((rows_per_chunk, S, 1, H), jnp.float32)] * NBUF
          + [pltpu.SemaphoreType.DMA] * NBUF,
    )
    def k(idx_hbm, table_hbm, out_hbm, tbl_v, ids_v, *bufs_sems):
        bufs, sems = bufs_sems[:NBUF], bufs_sems[NBUF:]
        wid = lax.axis_index("s") * NC + lax.axis_index("c")
        base = wid * n_per_w
        lane = lax.iota(jnp.int32, 16)

        pltpu.sync_copy(table_hbm, tbl_v)
        pltpu.sync_copy(idx_hbm.at[pl.ds(base, n_per_w)], ids_v)

        def construct(g, buf):
            goff = g * C

            @plsc.parallel_loop(0, C, unroll=2)
            def _(t):
                row = plsc.load_gather(
                    ids_v, [jnp.full((16,), goff + t, jnp.int32)]) << 8
                for j in range(16):
                    buf[t // S, t % S, 0, pl.ds(j * 16, 16)] = plsc.load_gather(
                        tbl_v, [row + (j * 16) + lane])

        def start_out(g, b):
            r0 = (base + g * C) // S
            pltpu.async_copy(
                bufs[b], out_hbm.at[pl.ds(r0, rows_per_chunk)], sems[b])

        def wait_out(b):
            pltpu.make_async_copy(
                bufs[b], out_hbm.at[pl.ds(base // S, rows_per_chunk)],
                sems[b]).wait()

        for b in range(NBUF):
            construct(b, bufs[b])
            start_out(b, b)

        def body(p, carry):
            g0 = NBUF * p + NBUF
            for b in range(NBUF):
                wait_out(b)
                construct(g0 + b, bufs[b])
                start_out(g0 + b, b)
            return carry

        lax.fori_loop(0, (nchunks - NBUF) // NBUF, body, 0)
        for b in range(NBUF):
            wait_out(b)

    return k(ids, table_flat)


def kernel(input_ids, table):
    BT = input_ids.size
    S = input_ids.shape[1]
    ids = input_ids.reshape(BT).astype(jnp.int32)
    out = _sc_lookup(ids, table.reshape(-1), BT, S)
    return out.reshape(*input_ids.shape, table.shape[1])


# speedup vs baseline: 1.0211x; 1.0211x over previous
import functools

import jax
import jax.numpy as jnp
from jax import lax
from jax.experimental import pallas as pl
from jax.experimental.pallas import tpu as pltpu
from jax.experimental.pallas import tpu_sc as plsc

H = 256
NC, NS = 2, 16
NW = NC * NS
C = 128
NBUF = 2


@functools.partial(jax.jit, static_argnums=(2, 3))
def _sc_lookup(ids, table_flat, BT, S):
    n_per_w = BT // NW
    nchunks = n_per_w // C
    rows_per_chunk = C // S
    mesh = plsc.VectorSubcoreMesh(
        core_axis_name="c", subcore_axis_name="s",
        num_cores=NC, num_subcores=NS,
    )

    @functools.partial(
        pl.kernel,
        out_type=jax.ShapeDtypeStruct((BT // S, S, 1, H), jnp.float32),
        mesh=mesh,
        compiler_params=pltpu.CompilerParams(
            needs_layout_passes=False, skip_device_barrier=True),
        scratch_types=[
            pltpu.VMEM((H * 3,), jnp.float32),
            pltpu.VMEM((n_per_w,), jnp.int32),
        ] + [pltpu.VMEM((rows_per_chunk, S, 1, H), jnp.float32)] * NBUF
          + [pltpu.SemaphoreType.DMA] * NBUF,
    )
    def k(idx_hbm, table_hbm, out_hbm, tbl_v, ids_v, *bufs_sems):
        bufs, sems = bufs_sems[:NBUF], bufs_sems[NBUF:]
        wid = lax.axis_index("s") * NC + lax.axis_index("c")
        base = wid * n_per_w
        lane = lax.iota(jnp.int32, 16)

        pltpu.sync_copy(table_hbm, tbl_v)
        pltpu.sync_copy(idx_hbm.at[pl.ds(base, n_per_w)], ids_v)

        def construct(g, buf):
            goff = g * C

            @plsc.parallel_loop(0, C, unroll=2)
            def _(t):
                row = plsc.load_gather(
                    ids_v, [jnp.full((16,), goff + t, jnp.int32)]) << 8
                for j in range(16):
                    buf[t // S, t % S, 0, pl.ds(j * 16, 16)] = plsc.load_gather(
                        tbl_v, [row + (j * 16) + lane])

        def start_out(g, b):
            r0 = (base + g * C) // S
            pltpu.async_copy(
                bufs[b], out_hbm.at[pl.ds(r0, rows_per_chunk)], sems[b])

        def wait_out(b):
            pltpu.make_async_copy(
                bufs[b], out_hbm.at[pl.ds(base // S, rows_per_chunk)],
                sems[b]).wait()

        for b in range(NBUF):
            construct(b, bufs[b])
            start_out(b, b)

        def body(p, carry):
            g0 = NBUF * p + NBUF
            for b in range(NBUF):
                wait_out(b)
                construct(g0 + b, bufs[b])
                start_out(g0 + b, b)
            return carry

        lax.fori_loop(0, (nchunks - NBUF) // NBUF, body, 0)
        for b in range(NBUF):
            wait_out(b)

    return k(ids, table_flat)


def kernel(input_ids, table):
    BT = input_ids.size
    S = input_ids.shape[1]
    ids = input_ids.reshape(BT).astype(jnp.int32)
    out = _sc_lookup(ids, table.reshape(-1), BT, S)
    return out.reshape(*input_ids.shape, table.shape[1])
